# baseline (speedup 1.0000x reference)
import functools

import jax
import jax.numpy as jnp
from jax import lax
from jax.experimental import pallas as pl
from jax.experimental.pallas import tpu as pltpu
from jax.experimental.pallas import tpu_sc as plsc

R = 4096
C = 100000
NS = 16
PER = R // NS
L = 16
CHUNK = 128


def _lane_shuffle(v, idx):
    dn = lax.GatherDimensionNumbers(
        offset_dims=(), collapsed_slice_dims=(0,), start_index_map=(0,))
    return lax.gather(v, idx[:, None], dn, (1,),
                      mode=lax.GatherScatterMode.PROMISE_IN_BOUNDS)


def _sc_logit_loss(logits_flat, labels):
    mesh = plsc.VectorSubcoreMesh(
        core_axis_name="c", subcore_axis_name="s", num_cores=1,
        num_subcores=NS)

    @functools.partial(
        pl.kernel,
        out_type=(jax.ShapeDtypeStruct((L,), jnp.float32),
                  jax.ShapeDtypeStruct((NS, L), jnp.float32)),
        mesh=mesh,
        scratch_types=[
            pltpu.VMEM((PER,), jnp.int32),
            pltpu.VMEM((CHUNK,), jnp.int32),
            pltpu.VMEM((CHUNK,), jnp.int32),
            pltpu.VMEM((CHUNK,), jnp.float32),
            pltpu.VMEM((CHUNK,), jnp.float32),
            pltpu.VMEM((L,), jnp.float32),
            pltpu.VMEM((NS, L), jnp.float32),
            pltpu.SemaphoreType.DMA,
        ],
    )
    def body(logits_hbm, labels_hbm, out_hbm, part_hbm,
             lab_v, idx_a, idx_b, val_a, val_b, stage_v, tmp_v, sem):
        s = lax.axis_index("s")
        base = s * PER
        pltpu.sync_copy(labels_hbm.at[pl.ds(base, PER)], lab_v)
        for j in range(PER // L):
            lab = lab_v[pl.ds(j * L, L)]
            rows = lax.iota(jnp.int32, L) + (base + j * L)
            flat = rows * C + lab
            if j < CHUNK // L:
                idx_a[pl.ds(j * L, L)] = flat
            else:
                idx_b[pl.ds((j - CHUNK // L) * L, L)] = flat
        cp_a = pltpu.async_copy(logits_hbm.at[idx_a], val_a, sem)
        cp_b = pltpu.async_copy(logits_hbm.at[idx_b], val_b, sem)
        cp_a.wait()
        cp_b.wait()
        acc = jnp.zeros((L,), jnp.float32)
        for j in range(CHUNK // L):
            acc = acc + val_a[pl.ds(j * L, L)] + val_b[pl.ds(j * L, L)]
        stage_v[...] = acc
        pltpu.sync_copy(stage_v, part_hbm.at[s])
        plsc.subcore_barrier()

        @pl.when(s == 0)
        def _():
            pltpu.sync_copy(part_hbm, tmp_v)
            tot = jnp.zeros((L,), jnp.float32)
            for r in range(NS):
                tot = tot + tmp_v[r]
            lane = lax.iota(jnp.int32, L)
            for sh in (8, 4, 2, 1):
                tot = tot + _lane_shuffle(tot, lane ^ sh)
            stage_v[...] = tot * (-1.0 / R)
            pltpu.sync_copy(stage_v, out_hbm)

    return body(logits_flat, labels)


def kernel(logits, labels):
    out, _ = _sc_logit_loss(logits.reshape(-1), labels.astype(jnp.int32))
    return out[0]


# speedup vs baseline: 2.4050x; 2.4050x over previous
import functools

import jax
import jax.numpy as jnp
from jax import lax
from jax.experimental import pallas as pl
from jax.experimental.pallas import tpu as pltpu
from jax.experimental.pallas import tpu_sc as plsc

R = 4096
C = 100000
NS = 16
PER = R // NS
L = 16
BLK = 128
BATCH = 16
TPP = 64


def _lane_shuffle(v, idx):
    dn = lax.GatherDimensionNumbers(
        offset_dims=(), collapsed_slice_dims=(0,), start_index_map=(0,))
    return lax.gather(v, idx[:, None], dn, (1,),
                      mode=lax.GatherScatterMode.PROMISE_IN_BOUNDS)


def _sc_logit_loss(logits, labels):
    mesh = plsc.VectorSubcoreMesh(
        core_axis_name="c", subcore_axis_name="s", num_cores=1,
        num_subcores=NS)

    @functools.partial(
        pl.kernel,
        out_type=(jax.ShapeDtypeStruct((L,), jnp.float32),
                  jax.ShapeDtypeStruct((NS, L), jnp.float32)),
        mesh=mesh,
        compiler_params=pltpu.CompilerParams(needs_layout_passes=False),
        scratch_types=[
            pltpu.VMEM((PER,), jnp.int32),
            pltpu.VMEM((TPP, 8, BLK), jnp.float32),
            pltpu.VMEM((L,), jnp.float32),
            pltpu.VMEM((NS, L), jnp.float32),
            pltpu.SemaphoreType.DMA,
        ],
    )
    def body(logits_hbm, labels_hbm, out_hbm, part_hbm,
             lab_v, blk_v, stage_v, tmp_v, sem):
        s = lax.axis_index("s")
        base = s * PER
        pltpu.sync_copy(labels_hbm.at[pl.ds(base, PER)], lab_v)

        def pass_body(p, carry):
            k0 = p * TPP
            for b in range(TPP // BATCH):
                lab16 = lab_v[pl.ds(k0 + b * BATCH, BATCH)]
                col0 = (lab16 // BLK) * BLK
                cps = []
                for j in range(BATCH):
                    t = b * BATCH + j
                    row0 = pl.multiple_of(base + k0 + (t // 8) * 8, 8)
                    cj = pl.multiple_of(col0[j], BLK)
                    cps.append(pltpu.async_copy(
                        logits_hbm.at[pl.ds(row0, 8), pl.ds(cj, BLK)],
                        blk_v.at[t], sem))
                for cp in cps:
                    cp.wait()
            acc_p = jnp.zeros((L,), jnp.float32)
            lane = lax.iota(jnp.int32, L)
            for c in range(TPP // L):
                lab16 = lab_v[pl.ds(k0 + c * L, L)]
                tiles = lane + c * L
                subs = lane % 8
                cols = lab16 % BLK
                acc_p = acc_p + plsc.load_gather(blk_v, [tiles, subs, cols])
            return carry + acc_p

        acc = lax.fori_loop(0, PER // TPP, pass_body,
                            jnp.zeros((L,), jnp.float32))
        stage_v[...] = acc
        pltpu.sync_copy(stage_v, part_hbm.at[s])
        plsc.subcore_barrier()

        @pl.when(s == 0)
        def _():
            pltpu.sync_copy(part_hbm, tmp_v)
            tot = jnp.zeros((L,), jnp.float32)
            for r in range(NS):
                tot = tot + tmp_v[r]
            lane = lax.iota(jnp.int32, L)
            for sh in (8, 4, 2, 1):
                tot = tot + _lane_shuffle(tot, lane ^ sh)
            stage_v[...] = tot * (-1.0 / R)
            pltpu.sync_copy(stage_v, out_hbm)

    return body(logits, labels)


def kernel(logits, labels):
    out, _ = _sc_logit_loss(logits, labels.astype(jnp.int32))
    return out[0]


# speedup vs baseline: 151.6332x; 63.0488x over previous
import functools

import jax
import jax.numpy as jnp
from jax import lax
from jax.experimental import pallas as pl
from jax.experimental.pallas import tpu as pltpu
from jax.experimental.pallas import tpu_sc as plsc

R = 4096
C = 100000
NS = 16
PER = R // NS
L = 16
G = 128


def _lane_shuffle(v, idx):
    dn = lax.GatherDimensionNumbers(
        offset_dims=(), collapsed_slice_dims=(0,), start_index_map=(0,))
    return lax.gather(v, idx[:, None], dn, (1,),
                      mode=lax.GatherScatterMode.PROMISE_IN_BOUNDS)


def _sc_logit_loss(logits_t, labels):
    mesh = plsc.VectorSubcoreMesh(
        core_axis_name="c", subcore_axis_name="s", num_cores=1,
        num_subcores=NS)

    @functools.partial(
        pl.kernel,
        out_type=(jax.ShapeDtypeStruct((L,), jnp.float32),
                  jax.ShapeDtypeStruct((NS, L), jnp.float32)),
        mesh=mesh,
        compiler_params=pltpu.CompilerParams(needs_layout_passes=False),
        scratch_types=[
            pltpu.VMEM((PER,), jnp.int32),
            pltpu.VMEM((G, G), jnp.float32),
            pltpu.VMEM((G, G), jnp.float32),
            pltpu.VMEM((L,), jnp.float32),
            pltpu.VMEM((NS, L), jnp.float32),
            pltpu.SemaphoreType.DMA,
        ],
    )
    def body(lt_hbm, labels_hbm, out_hbm, part_hbm,
             lab_v, blk_a, blk_b, stage_v, tmp_v, sem):
        s = lax.axis_index("s")
        base = s * PER
        pltpu.sync_copy(labels_hbm.at[pl.ds(base, PER)], lab_v)
        cp_a = pltpu.async_copy(
            lt_hbm.at[lab_v.at[pl.ds(0, G)], pl.ds(base, G)], blk_a, sem)
        cp_b = pltpu.async_copy(
            lt_hbm.at[lab_v.at[pl.ds(G, G)], pl.ds(base + G, G)], blk_b, sem)
        cp_a.wait()
        cp_b.wait()
        acc = jnp.zeros((L,), jnp.float32)
        lane = lax.iota(jnp.int32, L)
        for c in range(G // L):
            diag = lane + c * L
            acc = acc + plsc.load_gather(blk_a, [diag, diag])
            acc = acc + plsc.load_gather(blk_b, [diag, diag])
        stage_v[...] = acc
        pltpu.sync_copy(stage_v, part_hbm.at[s])
        plsc.subcore_barrier()

        @pl.when(s == 0)
        def _():
            pltpu.sync_copy(part_hbm, tmp_v)
            tot = jnp.zeros((L,), jnp.float32)
            for r in range(NS):
                tot = tot + tmp_v[r]
            for sh in (8, 4, 2, 1):
                tot = tot + _lane_shuffle(tot, lane ^ sh)
            stage_v[...] = tot * (-1.0 / R)
            pltpu.sync_copy(stage_v, out_hbm)

    return body(logits_t, labels)


def kernel(logits, labels):
    out, _ = _sc_logit_loss(logits.T, labels.astype(jnp.int32))
    return out[0]
